# Initial kernel scaffold; baseline (speedup 1.0000x reference)
import jax
import jax.numpy as jnp
from jax.experimental import pallas as pl


def kernel(hidden_states, W_gate, W_g, W_u, W_d):
    raise NotImplementedError("write your pallas kernel here")


# speedup vs baseline: 1.2470x; 1.2470x over previous
import functools

import jax
import jax.numpy as jnp
from jax.experimental import pallas as pl
from jax.experimental.pallas import tpu as pltpu


def _router_body(x_ref, wg_ref, logits_ref, comb_ref):
    x = x_ref[...]
    wg = wg_ref[...]
    logits = jax.lax.dot_general(
        x, wg, (((1,), (1,)), ((), ())),
        preferred_element_type=jnp.float32,
    )
    logits_ref[...] = logits
    m = jnp.max(logits, axis=1, keepdims=True)
    ex = jnp.exp(logits - m)
    p = ex / jnp.sum(ex, axis=1, keepdims=True)
    T, E = p.shape
    eidx = jax.lax.broadcasted_iota(jnp.int32, (T, E), 1)
    a1 = jnp.argmax(p, axis=1).astype(jnp.int32)
    m1 = jnp.max(p, axis=1)
    mask1 = eidx == a1[:, None]
    p2 = jnp.where(mask1, -1.0, p)
    a2 = jnp.argmax(p2, axis=1).astype(jnp.int32)
    m2 = jnp.max(p2, axis=1)
    denom = m1 + m2
    w1 = m1 / denom
    w2 = m2 / denom
    mask2 = eidx == a2[:, None]
    comb_ref[...] = (jnp.where(mask1, w1[:, None], 0.0)
                     + jnp.where(mask2, w2[:, None], 0.0))


def _moe_body(x_ref, wg_ref, wu_ref, wd_ref, comb_ref, out_ref, *, bt):
    e = pl.program_id(0)
    t = pl.program_id(1)
    xb = x_ref[...]
    wg = wg_ref[0]
    wu = wu_ref[0]
    wd = wd_ref[0]
    g = jax.lax.dot_general(xb, wg, (((1,), (1,)), ((), ())),
                            preferred_element_type=jnp.float32)
    u = jax.lax.dot_general(xb, wu, (((1,), (1,)), ((), ())),
                            preferred_element_type=jnp.float32)
    h = (g * jax.lax.logistic(g) * u).astype(jnp.bfloat16)
    y = jax.lax.dot_general(h, wd, (((1,), (1,)), ((), ())),
                            preferred_element_type=jnp.float32)
    c_all = comb_ref[pl.ds(t * bt, bt), :]
    eidx = jax.lax.broadcasted_iota(jnp.int32, c_all.shape, 1)
    c = jnp.sum(jnp.where(eidx == e, c_all, 0.0), axis=1)
    y = y * c[:, None]

    @pl.when(e == 0)
    def _init():
        out_ref[pl.ds(t * bt, bt), :] = y

    @pl.when(e != 0)
    def _acc():
        out_ref[pl.ds(t * bt, bt), :] += y


def kernel(hidden_states, W_gate, W_g, W_u, W_d):
    b, s, d = hidden_states.shape
    x = hidden_states.reshape(-1, d)
    T, D = x.shape
    E, F, _ = W_g.shape

    logits, comb = pl.pallas_call(
        _router_body,
        out_shape=(
            jax.ShapeDtypeStruct((T, E), jnp.float32),
            jax.ShapeDtypeStruct((T, E), jnp.float32),
        ),
    )(x, W_gate)

    BT = 256
    NT = T // BT
    x_bf = x.astype(jnp.bfloat16)
    wg_bf = W_g.astype(jnp.bfloat16)
    wu_bf = W_u.astype(jnp.bfloat16)
    wd_bf = W_d.astype(jnp.bfloat16)

    out = pl.pallas_call(
        functools.partial(_moe_body, bt=BT),
        grid=(E, NT),
        in_specs=[
            pl.BlockSpec((BT, D), lambda e, t: (t, 0)),
            pl.BlockSpec((1, F, D), lambda e, t: (e, 0, 0)),
            pl.BlockSpec((1, F, D), lambda e, t: (e, 0, 0)),
            pl.BlockSpec((1, D, F), lambda e, t: (e, 0, 0)),
            pl.BlockSpec((T, E), lambda e, t: (0, 0)),
        ],
        out_specs=pl.BlockSpec((T, D), lambda e, t: (0, 0)),
        out_shape=jax.ShapeDtypeStruct((T, D), jnp.float32),
    )(x_bf, wg_bf, wu_bf, wd_bf, comb)

    return out.reshape(b, s, d), logits.reshape(b, s, E)
